# baseline (speedup 1.0000x reference)
import functools

import jax
import jax.numpy as jnp
from jax import lax
from jax.experimental import pallas as pl
from jax.experimental.pallas import tpu as pltpu
from jax.experimental.pallas import tpu_sc as plsc

_NUM_CORES = 2
_NUM_SUBCORES = 16
_NUM_WORKERS = _NUM_CORES * _NUM_SUBCORES

_CHUNK = 512


@functools.cache
def _build(n_total: int, vocab: int, d: int):
    assert n_total % _NUM_WORKERS == 0
    n_per_w = n_total // _NUM_WORKERS
    chunk = min(_CHUNK, n_per_w)
    assert n_per_w % chunk == 0
    n_chunks = n_per_w // chunk

    mesh = plsc.VectorSubcoreMesh(core_axis_name="c", subcore_axis_name="s")

    @functools.partial(
        pl.kernel,
        out_type=jax.ShapeDtypeStruct((n_total, d), jnp.float32),
        mesh=mesh,
        scratch_types=[
            pltpu.VMEM((chunk,), jnp.int32),
            pltpu.VMEM((chunk, d), jnp.float32),
            pltpu.SemaphoreType.DMA,
        ],
        compiler_params=pltpu.CompilerParams(use_tc_tiling_on_sc=False),
    )
    def gather_kernel(idx_hbm, table_hbm, out_hbm, idx_v, rows_v, sem):
        wid = lax.axis_index("s") * _NUM_CORES + lax.axis_index("c")
        base_w = wid * n_per_w

        def step(i, carry):
            base = base_w + i * chunk
            pltpu.sync_copy(idx_hbm.at[pl.ds(base, chunk)], idx_v)
            pltpu.async_copy(table_hbm.at[idx_v], rows_v, sem).wait()
            pltpu.sync_copy(rows_v, out_hbm.at[pl.ds(base, chunk)])
            return carry

        lax.fori_loop(0, n_chunks, step, 0, unroll=False)

    return gather_kernel


def kernel(x, table):
    b, h = x.shape
    vocab, d = table.shape
    idx = x.reshape(b * h).astype(jnp.int32)
    out = _build(b * h, vocab, d)(idx, table)
    return out.reshape(b, h, d)


# speedup vs baseline: 1.0438x; 1.0438x over previous
import functools

import jax
import jax.numpy as jnp
from jax import lax
from jax.experimental import pallas as pl
from jax.experimental.pallas import tpu as pltpu
from jax.experimental.pallas import tpu_sc as plsc

_NUM_CORES = 2
_NUM_SUBCORES = 16
_NUM_WORKERS = _NUM_CORES * _NUM_SUBCORES

_CHUNK = 640


@functools.cache
def _build(n_total: int, vocab: int, d: int):
    assert n_total % _NUM_WORKERS == 0
    n_per_w = n_total // _NUM_WORKERS
    chunk = min(_CHUNK, n_per_w)
    assert n_per_w % chunk == 0
    n_chunks = n_per_w // chunk
    assert n_chunks % 2 == 0 and n_chunks >= 2

    mesh = plsc.VectorSubcoreMesh(core_axis_name="c", subcore_axis_name="s")

    @functools.partial(
        pl.kernel,
        out_type=jax.ShapeDtypeStruct((n_total, d), jnp.float32),
        mesh=mesh,
        scratch_types=[
            pltpu.VMEM((n_per_w,), jnp.int32),
            pltpu.VMEM((chunk, d), jnp.float32),
            pltpu.VMEM((chunk, d), jnp.float32),
            pltpu.SemaphoreType.DMA,
            pltpu.SemaphoreType.DMA,
            pltpu.SemaphoreType.DMA,
            pltpu.SemaphoreType.DMA,
        ],
        compiler_params=pltpu.CompilerParams(use_tc_tiling_on_sc=False),
    )
    def gather_kernel(idx_hbm, table_hbm, out_hbm, idx_v, rows0, rows1,
                      g0, g1, w0, w1):
        wid = lax.axis_index("s") * _NUM_CORES + lax.axis_index("c")
        base_w = wid * n_per_w
        rows = (rows0, rows1)
        gsem = (g0, g1)
        wsem = (w0, w1)

        pltpu.sync_copy(idx_hbm.at[pl.ds(base_w, n_per_w)], idx_v)

        def start_gather(j, b):
            pltpu.async_copy(
                table_hbm.at[idx_v.at[pl.ds(j * chunk, chunk)]],
                rows[b], gsem[b])

        def wait_gather(b):
            pltpu.make_async_copy(
                table_hbm.at[pl.ds(0, chunk)], rows[b], gsem[b]).wait()

        start_gather(0, 0)
        if n_chunks > 1:
            start_gather(1, 1)

        def group(g, carry):
            for b in range(2):
                i = g * 2 + b
                wait_gather(b)
                wb = pltpu.make_async_copy(
                    rows[b], out_hbm.at[pl.ds(base_w + i * chunk, chunk)],
                    wsem[b])
                wb.start()
                j = i + 2

                @pl.when(j < n_chunks)
                def _():
                    wb.wait()
                    start_gather(j, b)

            return carry

        lax.fori_loop(0, n_chunks // 2, group, 0, unroll=False)

        for b in range(2):
            i = n_chunks - 2 + b
            if i >= 0:
                pltpu.make_async_copy(
                    rows[b], out_hbm.at[pl.ds(base_w + i * chunk, chunk)],
                    wsem[b]).wait()

    return gather_kernel


def kernel(x, table):
    b, h = x.shape
    vocab, d = table.shape
    idx = x.reshape(b * h).astype(jnp.int32)
    out = _build(b * h, vocab, d)(idx, table)
    return out.reshape(b, h, d)


# speedup vs baseline: 1.2729x; 1.2195x over previous
import functools

import jax
import jax.numpy as jnp
from jax import lax
from jax.experimental import pallas as pl
from jax.experimental.pallas import tpu as pltpu
from jax.experimental.pallas import tpu_sc as plsc

_NUM_CORES = 2
_NUM_SUBCORES = 16
_NUM_WORKERS = _NUM_CORES * _NUM_SUBCORES

_CHUNK = 320


@functools.cache
def _build(n_total: int, vocab: int, dp: int):
    assert n_total % _NUM_WORKERS == 0
    n_per_w = n_total // _NUM_WORKERS
    chunk = min(_CHUNK, n_per_w)
    assert n_per_w % chunk == 0
    n_chunks = n_per_w // chunk
    assert n_chunks % 2 == 0 and n_chunks >= 2

    mesh = plsc.VectorSubcoreMesh(core_axis_name="c", subcore_axis_name="s")

    @functools.partial(
        pl.kernel,
        out_type=jax.ShapeDtypeStruct((n_total, dp), jnp.float32),
        mesh=mesh,
        scratch_types=[
            pltpu.VMEM((n_per_w,), jnp.int32),
            pltpu.VMEM((chunk, dp), jnp.float32),
            pltpu.VMEM((chunk, dp), jnp.float32),
            pltpu.SemaphoreType.DMA,
            pltpu.SemaphoreType.DMA,
            pltpu.SemaphoreType.DMA,
            pltpu.SemaphoreType.DMA,
        ],
    )
    def gather_kernel(idx_hbm, table_hbm, out_hbm, idx_v, rows0, rows1,
                      g0, g1, w0, w1):
        wid = lax.axis_index("s") * _NUM_CORES + lax.axis_index("c")
        base_w = wid * n_per_w
        rows = (rows0, rows1)
        gsem = (g0, g1)
        wsem = (w0, w1)

        pltpu.sync_copy(idx_hbm.at[pl.ds(base_w, n_per_w)], idx_v)

        def start_gather(j, b):
            pltpu.async_copy(
                table_hbm.at[idx_v.at[pl.ds(j * chunk, chunk)]],
                rows[b], gsem[b])

        def wait_gather(b):
            pltpu.make_async_copy(
                table_hbm.at[pl.ds(0, chunk)], rows[b], gsem[b]).wait()

        start_gather(0, 0)
        start_gather(1, 1)

        def group(g, carry):
            for b in range(2):
                i = g * 2 + b
                wait_gather(b)
                wb = pltpu.make_async_copy(
                    rows[b], out_hbm.at[pl.ds(base_w + i * chunk, chunk)],
                    wsem[b])
                wb.start()
                j = i + 2

                @pl.when(j < n_chunks)
                def _():
                    wb.wait()
                    start_gather(j, b)

            return carry

        lax.fori_loop(0, n_chunks // 2, group, 0, unroll=False)

        for b in range(2):
            i = n_chunks - 2 + b
            pltpu.make_async_copy(
                rows[b], out_hbm.at[pl.ds(base_w + i * chunk, chunk)],
                wsem[b]).wait()

    return gather_kernel


def kernel(x, table):
    b, h = x.shape
    vocab, d = table.shape
    dp = 128
    table_p = jnp.pad(table, ((0, 0), (0, dp - d)))
    idx = x.reshape(b * h).astype(jnp.int32)
    out_p = _build(b * h, vocab, dp)(idx, table_p)
    return out_p[:, :d].reshape(b, h, d)
